# Initial kernel scaffold; baseline (speedup 1.0000x reference)
import jax
import jax.numpy as jnp
from jax.experimental import pallas as pl


def kernel(p_coords, p_vals, p_b_idx, p_channels, dead_towers_mask, k_coords, k_vals, k_teams, k_b_idx, obj_status):
    raise NotImplementedError("write your pallas kernel here")


# speedup vs baseline: 2.4403x; 2.4403x over previous
import functools

import jax
import jax.numpy as jnp
import numpy as np
from jax.experimental import pallas as pl
from jax.experimental.pallas import tpu as pltpu

GRID = 64
MAP_SIZE = 15000.0
SIGMA = 1.5
SCALE = MAP_SIZE / GRID
R = int(1100 / SCALE)
NUM_CH = 16
B = 512
NB = 8

_TOWER_LOCATIONS = np.array([[8955,8510],[9767,10113],[11134,11207],[11593,11669],[13052,12612],[12611,13084],[5846,6396],[5048,4812],[3651,3696],[3210,3217],[2177,1807],[1748,2270],[4318,13875],[981,10441],[7943,13411],[1512,6699],[1169,4287],[1172,3583],[10481,13650],[11275,13657],[11275,13663],[13866,4505],[10504,1029],[13327,8226],[13624,10572],[6919,1483],[13599,11319],[4281,1253],[3468,1230],[13594,11319]], dtype=np.float32)
_TOWER_GRID = np.clip((_TOWER_LOCATIONS / SCALE).astype(np.int64), 0, GRID - 1)

_ks = int(2 * 4 * SIGMA + 1)
if _ks % 2 == 0:
    _ks += 1
_x = np.arange(_ks, dtype=np.float64) - _ks // 2
_k1d = np.exp(-_x ** 2 / (2 * SIGMA ** 2))
_kn1d = _k1d / _k1d.sum()
_A = np.zeros((GRID, GRID), np.float64)
for _i in range(GRID):
    for _j in range(GRID):
        if abs(_j - _i) <= _ks // 2:
            _A[_i, _j] = _kn1d[_ks // 2 + (_j - _i)]
_Af = _A.astype(np.float32)

_TP = np.zeros((30, GRID * GRID), np.float64)
_TD3 = np.zeros((30, GRID, GRID), np.float32)
_yy, _xx = np.meshgrid(np.arange(GRID), np.arange(GRID), indexing="ij")
for _t in range(30):
    _cx, _cy = int(_TOWER_GRID[_t, 0]), int(_TOWER_GRID[_t, 1])
    _TP[_t, _cy * GRID + _cx] = 1.0
    _disk = ((_xx - _cx) ** 2 + (_yy - _cy) ** 2) <= R * R
    _TD3[_t] = _disk.astype(np.float32)

_M30 = np.eye(30, dtype=np.float32)
_seen = {}
for _t in range(30):
    _cell = (int(_TOWER_GRID[_t, 0]), int(_TOWER_GRID[_t, 1]))
    if _cell in _seen:
        _M30[_t, _t] = 0.0
        _M30[_t, _seen[_cell]] = 1.0
        _TP[_t] = 0.0
    else:
        _seen[_cell] = _t

def _blur2d_flat(mat_flat64):
    out = np.zeros_like(mat_flat64)
    for r in range(mat_flat64.shape[0]):
        img = mat_flat64[r].reshape(GRID, GRID)
        img = _A.T @ img @ _A
        out[r] = img.reshape(-1)
    return out

_TPB = _blur2d_flat(_TP).astype(np.float32)
_drag = np.clip((np.array([10000.0, 5000.0]) / SCALE).astype(np.int64), 0, GRID - 1)
_baron = np.clip((np.array([5000.0, 10000.0]) / SCALE).astype(np.int64), 0, GRID - 1)
_OB = np.zeros((2, GRID * GRID), np.float64)
_OB[0, int(_drag[1]) * GRID + int(_drag[0])] = 1.0
_OB[1, int(_baron[1]) * GRID + int(_baron[0])] = 1.0
_OBB = _blur2d_flat(_OB).astype(np.float32)


def _tc_body(scat_ref, mask_ref, obj_ref, tpb_ref, td3_ref, obb_ref, a_ref,
             m30_ref, stamp_ref, conv_ref):
    alive = (mask_ref[...] <= 0.5).astype(jnp.float32)
    dead = 1.0 - alive
    objb = (obj_ref[...] > 0.5).astype(jnp.float32)
    m30 = m30_ref[...]
    alive_m = jnp.minimum(jax.lax.dot(alive, m30, preferred_element_type=jnp.float32), 1.0)
    dead_m = jnp.minimum(jax.lax.dot(dead, m30, preferred_element_type=jnp.float32), 1.0)
    tpb = tpb_ref[...]
    ch0 = jax.lax.dot(alive_m, tpb, preferred_element_type=jnp.float32)
    ch1 = jax.lax.dot(dead_m, tpb, preferred_element_type=jnp.float32)
    ch3 = jax.lax.dot(objb, obb_ref[...], preferred_element_type=jnp.float32)
    stamp_ref[0, :, :] = ch0
    stamp_ref[1, :, :] = ch1
    stamp_ref[2, :, :] = ch3
    a = a_ref[...]
    ch2 = jnp.zeros((NB, GRID, GRID), jnp.float32)
    for t in range(30):
        a_t = alive[:, t].reshape(NB, 1, 1)
        ch2 = jnp.maximum(ch2, a_t * td3_ref[t][None, :, :])
    canvas = jnp.concatenate([ch2[:, None], scat_ref[...]], axis=1)
    g = NB * 13
    t = jax.lax.dot(canvas.reshape(g * GRID, GRID), a,
                    preferred_element_type=jnp.float32)
    t = jnp.swapaxes(t.reshape(g, GRID, GRID), 1, 2)
    t = jax.lax.dot(t.reshape(g * GRID, GRID), a,
                    preferred_element_type=jnp.float32)
    t = jnp.swapaxes(t.reshape(g, GRID, GRID), 1, 2)
    conv_ref[...] = t.reshape(NB, 13, GRID, GRID)


@functools.partial(jax.jit, static_argnames=("interpret",))
def _tc_stage(scat, dead_towers_mask, obj_status, interpret=False):
    grid = (B // NB,)
    stamp, conv = pl.pallas_call(
        _tc_body,
        grid=grid,
        in_specs=[
            pl.BlockSpec((NB, 12, GRID, GRID), lambda i: (i, 0, 0, 0)),
            pl.BlockSpec((NB, 30), lambda i: (i, 0)),
            pl.BlockSpec((NB, 2), lambda i: (i, 0)),
            pl.BlockSpec((30, GRID * GRID), lambda i: (0, 0)),
            pl.BlockSpec((30, GRID, GRID), lambda i: (0, 0, 0)),
            pl.BlockSpec((2, GRID * GRID), lambda i: (0, 0)),
            pl.BlockSpec((GRID, GRID), lambda i: (0, 0)),
            pl.BlockSpec((30, 30), lambda i: (0, 0)),
        ],
        out_specs=[
            pl.BlockSpec((3, NB, GRID * GRID), lambda i: (0, i, 0)),
            pl.BlockSpec((NB, 13, GRID, GRID), lambda i: (i, 0, 0, 0)),
        ],
        out_shape=[
            jax.ShapeDtypeStruct((3, B, GRID * GRID), jnp.float32),
            jax.ShapeDtypeStruct((B, 13, GRID, GRID), jnp.float32),
        ],
        interpret=interpret,
    )(scat, dead_towers_mask, obj_status, jnp.asarray(_TPB), jnp.asarray(_TD3),
      jnp.asarray(_OBB), jnp.asarray(_Af), jnp.asarray(_M30))
    st = stamp.reshape(3, B, 1, GRID, GRID)
    out = jnp.concatenate(
        [st[0], st[1], conv[:, 0:1], st[2], conv[:, 1:13]], axis=1)
    return out


def _scatter_canvas(p_coords, p_vals, p_b_idx, p_channels, k_coords, k_vals, k_teams, k_b_idx):
    scat = jnp.zeros((B, 12, GRID, GRID), jnp.float32)
    kg = jnp.clip((k_coords / SCALE).astype(jnp.int32), 0, GRID - 1)
    scat = scat.at[k_b_idx, k_teams.astype(jnp.int32), kg[:, 1], kg[:, 0]].add(k_vals)
    pg = jnp.clip((p_coords / SCALE).astype(jnp.int32), 0, GRID - 1)
    scat = scat.at[p_b_idx, 2 + p_channels.astype(jnp.int32), pg[:, 1], pg[:, 0]].add(p_vals)
    return scat


def kernel(p_coords, p_vals, p_b_idx, p_channels, dead_towers_mask, k_coords, k_vals, k_teams, k_b_idx, obj_status):
    scat = _scatter_canvas(p_coords, p_vals, p_b_idx, p_channels, k_coords, k_vals, k_teams, k_b_idx)
    return _tc_stage(scat, dead_towers_mask, obj_status)


# speedup vs baseline: 3.1819x; 1.3039x over previous
import functools

import jax
import jax.numpy as jnp
import numpy as np
from jax import lax
from jax.experimental import pallas as pl
from jax.experimental.pallas import tpu as pltpu
from jax.experimental.pallas import tpu_sc as plsc

GRID = 64
MAP_SIZE = 15000.0
SIGMA = 1.5
SCALE = MAP_SIZE / GRID
R = int(1100 / SCALE)
NUM_CH = 16
B = 512
NB = 8

_TOWER_LOCATIONS = np.array([[8955,8510],[9767,10113],[11134,11207],[11593,11669],[13052,12612],[12611,13084],[5846,6396],[5048,4812],[3651,3696],[3210,3217],[2177,1807],[1748,2270],[4318,13875],[981,10441],[7943,13411],[1512,6699],[1169,4287],[1172,3583],[10481,13650],[11275,13657],[11275,13663],[13866,4505],[10504,1029],[13327,8226],[13624,10572],[6919,1483],[13599,11319],[4281,1253],[3468,1230],[13594,11319]], dtype=np.float32)
_TOWER_GRID = np.clip((_TOWER_LOCATIONS / SCALE).astype(np.int64), 0, GRID - 1)

_ks = int(2 * 4 * SIGMA + 1)
if _ks % 2 == 0:
    _ks += 1
_x = np.arange(_ks, dtype=np.float64) - _ks // 2
_k1d = np.exp(-_x ** 2 / (2 * SIGMA ** 2))
_kn1d = _k1d / _k1d.sum()
_A = np.zeros((GRID, GRID), np.float64)
for _i in range(GRID):
    for _j in range(GRID):
        if abs(_j - _i) <= _ks // 2:
            _A[_i, _j] = _kn1d[_ks // 2 + (_j - _i)]
_Af = _A.astype(np.float32)

_TP = np.zeros((30, GRID * GRID), np.float64)
_TD3 = np.zeros((30, GRID, GRID), np.float32)
_yy, _xx = np.meshgrid(np.arange(GRID), np.arange(GRID), indexing="ij")
for _t in range(30):
    _cx, _cy = int(_TOWER_GRID[_t, 0]), int(_TOWER_GRID[_t, 1])
    _TP[_t, _cy * GRID + _cx] = 1.0
    _disk = ((_xx - _cx) ** 2 + (_yy - _cy) ** 2) <= R * R
    _TD3[_t] = _disk.astype(np.float32)

_M30 = np.eye(30, dtype=np.float32)
_seen = {}
for _t in range(30):
    _cell = (int(_TOWER_GRID[_t, 0]), int(_TOWER_GRID[_t, 1]))
    if _cell in _seen:
        _M30[_t, _t] = 0.0
        _M30[_t, _seen[_cell]] = 1.0
        _TP[_t] = 0.0
    else:
        _seen[_cell] = _t

def _blur2d_flat(mat_flat64):
    out = np.zeros_like(mat_flat64)
    for r in range(mat_flat64.shape[0]):
        img = mat_flat64[r].reshape(GRID, GRID)
        img = _A.T @ img @ _A
        out[r] = img.reshape(-1)
    return out

_TPB = _blur2d_flat(_TP).astype(np.float32)
_drag = np.clip((np.array([10000.0, 5000.0]) / SCALE).astype(np.int64), 0, GRID - 1)
_baron = np.clip((np.array([5000.0, 10000.0]) / SCALE).astype(np.int64), 0, GRID - 1)
_OB = np.zeros((2, GRID * GRID), np.float64)
_OB[0, int(_drag[1]) * GRID + int(_drag[0])] = 1.0
_OB[1, int(_baron[1]) * GRID + int(_baron[0])] = 1.0
_OBB = _blur2d_flat(_OB).astype(np.float32)


def _tc_body(scat_ref, mask_ref, obj_ref, tpb_ref, td3_ref, obb_ref, a_ref,
             m30_ref, stamp_ref, conv_ref):
    alive = (mask_ref[...] <= 0.5).astype(jnp.float32)
    dead = 1.0 - alive
    objb = (obj_ref[...] > 0.5).astype(jnp.float32)
    m30 = m30_ref[...]
    alive_m = jnp.minimum(jax.lax.dot(alive, m30, preferred_element_type=jnp.float32), 1.0)
    dead_m = jnp.minimum(jax.lax.dot(dead, m30, preferred_element_type=jnp.float32), 1.0)
    tpb = tpb_ref[...]
    ch0 = jax.lax.dot(alive_m, tpb, preferred_element_type=jnp.float32)
    ch1 = jax.lax.dot(dead_m, tpb, preferred_element_type=jnp.float32)
    ch3 = jax.lax.dot(objb, obb_ref[...], preferred_element_type=jnp.float32)
    stamp_ref[0, :, :] = ch0
    stamp_ref[1, :, :] = ch1
    stamp_ref[2, :, :] = ch3
    a = a_ref[...]
    ch2 = jnp.zeros((NB, GRID, GRID), jnp.float32)
    for t in range(30):
        a_t = alive[:, t].reshape(NB, 1, 1)
        ch2 = jnp.maximum(ch2, a_t * td3_ref[t][None, :, :])
    canvas = jnp.concatenate([ch2[:, None], scat_ref[...]], axis=1)
    g = NB * 13
    t = jax.lax.dot(canvas.reshape(g * GRID, GRID), a,
                    preferred_element_type=jnp.float32)
    t = jnp.swapaxes(t.reshape(g, GRID, GRID), 1, 2)
    t = jax.lax.dot(t.reshape(g * GRID, GRID), a,
                    preferred_element_type=jnp.float32)
    t = jnp.swapaxes(t.reshape(g, GRID, GRID), 1, 2)
    conv_ref[...] = t.reshape(NB, 13, GRID, GRID)


@functools.partial(jax.jit, static_argnames=("interpret",))
def _tc_stage(scat, dead_towers_mask, obj_status, interpret=False):
    grid = (B // NB,)
    stamp, conv = pl.pallas_call(
        _tc_body,
        grid=grid,
        in_specs=[
            pl.BlockSpec((NB, 12, GRID, GRID), lambda i: (i, 0, 0, 0)),
            pl.BlockSpec((NB, 30), lambda i: (i, 0)),
            pl.BlockSpec((NB, 2), lambda i: (i, 0)),
            pl.BlockSpec((30, GRID * GRID), lambda i: (0, 0)),
            pl.BlockSpec((30, GRID, GRID), lambda i: (0, 0, 0)),
            pl.BlockSpec((2, GRID * GRID), lambda i: (0, 0)),
            pl.BlockSpec((GRID, GRID), lambda i: (0, 0)),
            pl.BlockSpec((30, 30), lambda i: (0, 0)),
        ],
        out_specs=[
            pl.BlockSpec((3, NB, GRID * GRID), lambda i: (0, i, 0)),
            pl.BlockSpec((NB, 13, GRID, GRID), lambda i: (i, 0, 0, 0)),
        ],
        out_shape=[
            jax.ShapeDtypeStruct((3, B, GRID * GRID), jnp.float32),
            jax.ShapeDtypeStruct((B, 13, GRID, GRID), jnp.float32),
        ],
        interpret=interpret,
    )(scat, dead_towers_mask, obj_status, jnp.asarray(_TPB), jnp.asarray(_TD3),
      jnp.asarray(_OBB), jnp.asarray(_Af), jnp.asarray(_M30))
    st = stamp.reshape(3, B, 1, GRID, GRID)
    out = jnp.concatenate(
        [st[0], st[1], conv[:, 0:1], st[2], conv[:, 1:13]], axis=1)
    return out


N_P = 425984
N_K = 131072
N_ALL = N_P + N_K
ROWS_ALL = N_ALL // 128
PW_P = N_P // 32
PW_K = N_K // 32
S_WIN = 524288
N_WIN_PER_SC = 24
TEC_ROWS = ROWS_ALL // 16
FLUSH_W = S_WIN // 16
ZN = 16384

_sc_mesh = plsc.VectorSubcoreMesh(core_axis_name="c", subcore_axis_name="s")


@functools.partial(
    pl.kernel,
    out_type=jax.ShapeDtypeStruct((ROWS_ALL, 128), jnp.int32),
    mesh=_sc_mesh,
    scratch_types=[
        pltpu.VMEM((PW_P // 128, 128), jnp.float32),
        pltpu.VMEM((PW_P // 128, 128), jnp.float32),
        pltpu.VMEM((PW_P // 128, 128), jnp.int32),
        pltpu.VMEM((PW_P // 128, 128), jnp.int32),
        pltpu.VMEM((PW_P // 128, 128), jnp.int32),
    ],
)
def _sc_prep(px, py, pb, pc, kx, ky, kb, kt, idx_out, xb, yb, bb, cb, ob):
    wid = lax.axis_index("s") * 2 + lax.axis_index("c")

    def section(xs, ys, bs, cs, rows, src_row0, dst_row0, coff):
        pltpu.sync_copy(xs.at[pl.ds(src_row0, rows)], xb.at[pl.ds(0, rows)])
        pltpu.sync_copy(ys.at[pl.ds(src_row0, rows)], yb.at[pl.ds(0, rows)])
        pltpu.sync_copy(bs.at[pl.ds(src_row0, rows)], bb.at[pl.ds(0, rows)])
        pltpu.sync_copy(cs.at[pl.ds(src_row0, rows)], cb.at[pl.ds(0, rows)])

        def row_body(r, _):
            for u in range(8):
                sl = pl.ds(u * 16, 16)
                xi = jnp.clip((xb[r, sl] / SCALE).astype(jnp.int32), 0, GRID - 1)
                yi = jnp.clip((yb[r, sl] / SCALE).astype(jnp.int32), 0, GRID - 1)
                c = cb[r, sl] + coff
                flat = bb[r, sl] * (12 * 4096) + c * 4096 + yi * 64 + xi
                ob[r, sl] = flat
            return 0

        lax.fori_loop(0, rows, row_body, 0)
        pltpu.sync_copy(ob.at[pl.ds(0, rows)], idx_out.at[pl.ds(dst_row0, rows)])

    section(px, py, pb, pc, PW_P // 128, wid * (PW_P // 128), wid * (PW_P // 128), 2)
    section(kx, ky, kb, kt, PW_K // 128, wid * (PW_K // 128),
            N_P // 128 + wid * (PW_K // 128), 0)


@functools.partial(
    pl.kernel,
    out_type=jax.ShapeDtypeStruct((B * 12 * GRID * GRID,), jnp.float32),
    mesh=_sc_mesh,
    scratch_types=[
        pltpu.VMEM((TEC_ROWS, 128), jnp.int32),
        pltpu.VMEM((TEC_ROWS, 128), jnp.float32),
        pltpu.VMEM((ZN,), jnp.float32),
        pltpu.VMEM_SHARED((S_WIN,), jnp.float32),
        pltpu.SemaphoreType.DMA,
    ],
)
def _sc_scatter(idx_hbm, val_hbm, out_hbm, idxb, valb, zbuf, win, sem):
    cid = lax.axis_index("c")
    sid = lax.axis_index("s")
    iota = lax.iota(jnp.int32, 16)
    zeros16 = jnp.zeros((16,), jnp.float32)

    def zinit(i, _):
        for u in range(16):
            zbuf[pl.ds((i * 16 + u) * 16, 16)] = zeros16
        return 0

    lax.fori_loop(0, ZN // 256, zinit, 0)

    def round_body(r, _):
        base = (cid * N_WIN_PER_SC + r) * S_WIN
        for j in range(FLUSH_W // ZN):
            pltpu.sync_copy(zbuf, win.at[pl.ds(sid * FLUSH_W + j * ZN, ZN)])
        plsc.subcore_barrier()
        pltpu.sync_copy(idx_hbm.at[pl.ds(sid * TEC_ROWS, TEC_ROWS)], idxb)
        pltpu.sync_copy(val_hbm.at[pl.ds(sid * TEC_ROWS, TEC_ROWS)], valb)

        def xform_fire(it, _):
            for u in range(8):
                row = it * 8 + u
                for v8 in range(8):
                    sl = pl.ds(v8 * 16, 16)
                    fidx = idxb[row, sl]
                    v = valb[row, sl]
                    m = jnp.logical_and(fidx >= base, fidx < base + S_WIN)
                    trash = (row * 128 + v8 * 16 + iota) & (ZN - 1)
                    idxb[row, sl] = jnp.where(m, fidx - base, trash)
                    valb[row, sl] = jnp.where(m, v, 0.0)
            hs = []
            for u in range(8):
                row = it * 8 + u
                hs.append(pltpu.async_copy(valb.at[row], win.at[idxb.at[row]], sem,
                                           add=True))
            for h in hs:
                h.wait()
            return 0

        lax.fori_loop(0, TEC_ROWS // 8, xform_fire, 0)
        plsc.subcore_barrier()
        pltpu.sync_copy(win.at[pl.ds(sid * FLUSH_W, FLUSH_W)],
                        out_hbm.at[pl.ds(base + sid * FLUSH_W, FLUSH_W)])
        return 0

    lax.fori_loop(0, N_WIN_PER_SC, round_body, 0)


def _scatter_canvas(p_coords, p_vals, p_b_idx, p_channels, k_coords, k_vals, k_teams, k_b_idx):
    P = p_vals.shape[0]
    K = k_vals.shape[0]
    padp = N_P - P
    padk = N_K - K
    px = jnp.concatenate([p_coords[:, 0], jnp.zeros((padp,), jnp.float32)])
    py = jnp.concatenate([p_coords[:, 1], jnp.zeros((padp,), jnp.float32)])
    pb = jnp.concatenate([p_b_idx.astype(jnp.int32),
                          jnp.arange(padp, dtype=jnp.int32) % B])
    pc = jnp.concatenate([p_channels.astype(jnp.int32), jnp.zeros((padp,), jnp.int32)])
    pv = jnp.concatenate([p_vals, jnp.zeros((padp,), jnp.float32)])
    kx = jnp.concatenate([k_coords[:, 0], jnp.zeros((padk,), jnp.float32)])
    ky = jnp.concatenate([k_coords[:, 1], jnp.zeros((padk,), jnp.float32)])
    kb = jnp.concatenate([k_b_idx.astype(jnp.int32),
                          jnp.arange(padk, dtype=jnp.int32) % B])
    kt = jnp.concatenate([k_teams.astype(jnp.int32), jnp.zeros((padk,), jnp.int32)])
    kv = jnp.concatenate([k_vals, jnp.zeros((padk,), jnp.float32)])
    r2 = lambda a: a.reshape(a.shape[0] // 128, 128)
    idx2 = _sc_prep(r2(px), r2(py), r2(pb), r2(pc), r2(kx), r2(ky), r2(kb), r2(kt))
    val2 = jnp.concatenate([pv, kv]).reshape(ROWS_ALL, 128)
    scat_flat = _sc_scatter(idx2, val2)
    return scat_flat.reshape(B, 12, GRID, GRID)


def kernel(p_coords, p_vals, p_b_idx, p_channels, dead_towers_mask, k_coords, k_vals, k_teams, k_b_idx, obj_status):
    scat = _scatter_canvas(p_coords, p_vals, p_b_idx, p_channels, k_coords, k_vals, k_teams, k_b_idx)
    return _tc_stage(scat, dead_towers_mask, obj_status)
